# baseline (speedup 1.0000x reference)
import functools

import jax
import jax.numpy as jnp
from jax import lax
from jax.experimental import pallas as pl
from jax.experimental.pallas import tpu as pltpu
from jax.experimental.pallas import tpu_sc as plsc

NC = 2
NS = 16
NW = NC * NS
L = 16
CHUNK = 128

N_NODES = 10000
N_PAD = 10112
RPT = N_PAD // NS
CORE0_FRAC = 0.50


def _sc_mesh():
    return plsc.VectorSubcoreMesh(core_axis_name="c", subcore_axis_name="s")


def _make_deg_kernel(kt, k0, k1):
    @functools.partial(
        pl.kernel,
        mesh=_sc_mesh(),
        out_type=jax.ShapeDtypeStruct((NW, N_PAD), jnp.float32),
        compiler_params=pltpu.CompilerParams(needs_layout_passes=False),
        scratch_types=[
            pltpu.VMEM((kt, CHUNK), jnp.int32),
            pltpu.VMEM((N_PAD,), jnp.float32),
        ],
    )
    def deg_kernel(dst_hbm, out_hbm, dst_v, deg_v):
        c = lax.axis_index("c")
        s = lax.axis_index("s")
        wid = c * NS + s
        pltpu.sync_copy(dst_hbm.at[wid], dst_v)

        zero = jnp.zeros((L,), jnp.float32)

        def zbody(i, carry):
            deg_v[pl.ds(i * L, L)] = zero
            return carry

        lax.fori_loop(0, N_PAD // L, zbody, 0)

        ones = jnp.ones((L,), jnp.float32)
        groups = CHUNK // L

        def body(i, carry):
            j = i // groups
            k = i % groups
            idx = dst_v[j, pl.ds(k * L, L)]
            plsc.addupdate_scatter(deg_v, [idx], ones)
            return carry

        kc = jnp.where(c == 0, k0, k1)
        lax.fori_loop(0, kc * groups, body, 0)
        pltpu.sync_copy(deg_v, out_hbm.at[wid])

    return deg_kernel


NBUF = 2
PAD_CHUNKS = NBUF


def _make_scatter_kernel(k0, k1, d):
    kt = max(k0, k1) + PAD_CHUNKS

    @functools.partial(
        pl.kernel,
        mesh=_sc_mesh(),
        out_type=jax.ShapeDtypeStruct((NC, N_PAD, d), jnp.float32),
        compiler_params=pltpu.CompilerParams(needs_layout_passes=False,
                                             use_tc_tiling_on_sc=False),
        scratch_types=[
            pltpu.VMEM((kt, CHUNK), jnp.int32),
            pltpu.VMEM((kt, CHUNK), jnp.int32),
            pltpu.VMEM((NBUF, CHUNK, d), jnp.float32),
            pltpu.VMEM_SHARED((N_PAD, d), jnp.float32),
        ] + [pltpu.SemaphoreType.DMA] * NBUF,
    )
    def sc_kernel(g_hbm, src_hbm, dst_hbm, zero_hbm, out_hbm,
                  src_v, dst_v, rows_v, acc_sh, *gsem):
        c = lax.axis_index("c")
        s = lax.axis_index("s")
        wid = c * NS + s
        row0 = s * RPT

        pltpu.sync_copy(zero_hbm.at[pl.ds(row0, RPT)],
                        acc_sh.at[pl.ds(row0, RPT)])
        pltpu.sync_copy(src_hbm.at[wid], src_v)
        pltpu.sync_copy(dst_hbm.at[wid], dst_v)
        plsc.subcore_barrier()

        for b in range(NBUF):
            pltpu.async_copy(g_hbm.at[src_v.at[b]], rows_v.at[b], gsem[b],
                             priority=1)

        def body(i, carry):
            j0 = NBUF * i
            for b in range(NBUF):
                jb = j0 + b
                pltpu.make_async_copy(g_hbm.at[src_v.at[jb]], rows_v.at[b],
                                      gsem[b]).wait()
                pltpu.sync_copy(rows_v.at[b], acc_sh.at[dst_v.at[jb]],
                                add=True)
                pltpu.async_copy(g_hbm.at[src_v.at[jb + NBUF]], rows_v.at[b],
                                 gsem[b], priority=1)
            return carry

        n_rounds = jnp.where(c == 0, k0 // NBUF, k1 // NBUF)
        lax.fori_loop(0, n_rounds, body, 0)
        for b in range(NBUF):
            pltpu.make_async_copy(g_hbm.at[src_v.at[b]], rows_v.at[b],
                                  gsem[b]).wait()

        plsc.subcore_barrier()
        pltpu.sync_copy(acc_sh.at[pl.ds(row0, RPT)],
                        out_hbm.at[c, pl.ds(row0, RPT)])

    return sc_kernel


RB = 1024
GRID = -(-N_NODES // RB)


def _tc_encode1(deg_parts, x, w1):
    def body(dp_ref, x_ref, w_ref, g_ref, dis_ref):
        deg = jnp.sum(dp_ref[...], axis=0) + 1.0
        dis = lax.rsqrt(deg)[:, None]
        h = jnp.dot(x_ref[...], w_ref[...],
                    preferred_element_type=jnp.float32)
        g_ref[...] = dis * h
        dis_ref[...] = dis

    return pl.pallas_call(
        body,
        grid=(GRID,),
        in_specs=[
            pl.BlockSpec((NW, RB), lambda i: (0, i)),
            pl.BlockSpec((RB, 128), lambda i: (i, 0)),
            pl.BlockSpec((128, 64), lambda i: (0, 0)),
        ],
        out_specs=[
            pl.BlockSpec((RB, 64), lambda i: (i, 0)),
            pl.BlockSpec((RB, 1), lambda i: (i, 0)),
        ],
        out_shape=[
            jax.ShapeDtypeStruct((N_NODES, 64), jnp.float32),
            jax.ShapeDtypeStruct((N_NODES, 1), jnp.float32),
        ],
    )(deg_parts, x, w1)


def _tc_encode2(disv, acc1, g1, b1, w2):
    def body(dis_ref, acc_ref, g1_ref, b_ref, w_ref, out_ref):
        dis = dis_ref[...]
        ssum = acc_ref[0] + acc_ref[1] + g1_ref[...]
        h1 = jnp.maximum(dis * ssum + b_ref[...], 0.0)
        h2 = jnp.dot(h1, w_ref[...], preferred_element_type=jnp.float32)
        out_ref[...] = dis * h2

    return pl.pallas_call(
        body,
        grid=(GRID,),
        in_specs=[
            pl.BlockSpec((RB, 1), lambda i: (i, 0)),
            pl.BlockSpec((NC, RB, 64), lambda i: (0, i, 0)),
            pl.BlockSpec((RB, 64), lambda i: (i, 0)),
            pl.BlockSpec((1, 64), lambda i: (0, 0)),
            pl.BlockSpec((64, 32), lambda i: (0, 0)),
        ],
        out_specs=pl.BlockSpec((RB, 32), lambda i: (i, 0)),
        out_shape=jax.ShapeDtypeStruct((N_NODES, 32), jnp.float32),
    )(disv, acc1, g1, b1, w2)


def _tc_decode(disv, acc2, g2, b2, wd1, bd1, wd2, bd2):
    def body(dis_ref, acc_ref, g2_ref, b2_ref, wd1_ref, bd1_ref, wd2_ref,
             bd2_ref, out_ref):
        dis = dis_ref[...]
        ssum = acc_ref[0] + acc_ref[1] + g2_ref[...]
        z = jnp.maximum(dis * ssum + b2_ref[...], 0.0)
        dlat = jnp.dot(z, wd1_ref[...], preferred_element_type=jnp.float32)
        dlat = jnp.maximum(dlat + bd1_ref[...], 0.0)
        logits = jnp.dot(dlat, wd2_ref[...],
                         preferred_element_type=jnp.float32)
        out_ref[...] = jax.nn.sigmoid(logits + bd2_ref[...])

    return pl.pallas_call(
        body,
        grid=(GRID,),
        in_specs=[
            pl.BlockSpec((RB, 1), lambda i: (i, 0)),
            pl.BlockSpec((NC, RB, 32), lambda i: (0, i, 0)),
            pl.BlockSpec((RB, 32), lambda i: (i, 0)),
            pl.BlockSpec((1, 32), lambda i: (0, 0)),
            pl.BlockSpec((32, 64), lambda i: (0, 0)),
            pl.BlockSpec((1, 64), lambda i: (0, 0)),
            pl.BlockSpec((64, 128), lambda i: (0, 0)),
            pl.BlockSpec((1, 128), lambda i: (0, 0)),
        ],
        out_specs=pl.BlockSpec((RB, 128), lambda i: (i, 0)),
        out_shape=jax.ShapeDtypeStruct((N_NODES, 128), jnp.float32),
    )(disv, acc2, g2, b2, wd1, bd1, wd2, bd2)


def kernel(x, edge_index, W1, b1, W2, b2, Wd1, bd1, Wd2, bd2):
    e = edge_index.shape[1]
    src = edge_index[0].astype(jnp.int32)
    dst = edge_index[1].astype(jnp.int32)

    chunks_tot = -(-e // CHUNK)
    k0 = int(round(chunks_tot * CORE0_FRAC / NS))
    k0 = max(NBUF, -(-k0 // NBUF) * NBUF)
    k1 = -(-(chunks_tot - NS * k0) // (NS * NBUF)) * NBUF
    k1 = max(NBUF, k1)
    kt = max(k0, k1) + PAD_CHUNKS

    cap0 = NS * k0 * CHUNK
    cap_tot = NS * (k0 + k1) * CHUNK
    src_f = jnp.concatenate([src, jnp.zeros((cap_tot - e,), jnp.int32)])
    dst_f = jnp.concatenate([dst, jnp.full((cap_tot - e,), N_NODES,
                                           jnp.int32)])

    def shard(flat, kc, fill):
        m = flat.reshape(NS, kc, CHUNK)
        padc = jnp.full((NS, kt - kc, CHUNK), fill, jnp.int32)
        return jnp.concatenate([m, padc], axis=1)

    src_all = jnp.concatenate([shard(src_f[:cap0], k0, 0),
                               shard(src_f[cap0:], k1, 0)], axis=0)
    dst_all = jnp.concatenate([shard(dst_f[:cap0], k0, N_NODES),
                               shard(dst_f[cap0:], k1, N_NODES)], axis=0)

    deg_parts = _make_deg_kernel(kt, k0, k1)(dst_all)

    g1, disv = _tc_encode1(deg_parts, x, W1)

    zeros64 = jnp.zeros((N_PAD, 64), jnp.float32)
    acc1 = _make_scatter_kernel(k0, k1, 64)(g1, src_all, dst_all, zeros64)

    g2 = _tc_encode2(disv, acc1[:, :N_NODES, :], g1, b1.reshape(1, 64), W2)

    zeros32 = jnp.zeros((N_PAD, 32), jnp.float32)
    acc2 = _make_scatter_kernel(k0, k1, 32)(g2, src_all, dst_all, zeros32)

    return _tc_decode(disv, acc2[:, :N_NODES, :], g2, b2.reshape(1, 32),
                      Wd1, bd1.reshape(1, 64), Wd2, bd2.reshape(1, 128))


# speedup vs baseline: 1.0301x; 1.0301x over previous
import functools

import jax
import jax.numpy as jnp
from jax import lax
from jax.experimental import pallas as pl
from jax.experimental.pallas import tpu as pltpu
from jax.experimental.pallas import tpu_sc as plsc

NC = 2
NS = 16
NW = NC * NS
L = 16
CHUNK = 128

N_NODES = 10000
N_PAD = 10112
RPT = N_PAD // NS
CORE0_FRAC = 0.63


def _sc_mesh():
    return plsc.VectorSubcoreMesh(core_axis_name="c", subcore_axis_name="s")


def _make_deg_kernel(kt, k0, k1):
    @functools.partial(
        pl.kernel,
        mesh=_sc_mesh(),
        out_type=jax.ShapeDtypeStruct((NW, N_PAD), jnp.float32),
        compiler_params=pltpu.CompilerParams(needs_layout_passes=False),
        scratch_types=[
            pltpu.VMEM((kt, CHUNK), jnp.int32),
            pltpu.VMEM((N_PAD,), jnp.float32),
        ],
    )
    def deg_kernel(dst_hbm, out_hbm, dst_v, deg_v):
        c = lax.axis_index("c")
        s = lax.axis_index("s")
        wid = c * NS + s
        pltpu.sync_copy(dst_hbm.at[wid], dst_v)

        zero = jnp.zeros((L,), jnp.float32)

        def zbody(i, carry):
            deg_v[pl.ds(i * L, L)] = zero
            return carry

        lax.fori_loop(0, N_PAD // L, zbody, 0)

        ones = jnp.ones((L,), jnp.float32)
        groups = CHUNK // L

        def body(i, carry):
            j = i // groups
            k = i % groups
            idx = dst_v[j, pl.ds(k * L, L)]
            plsc.addupdate_scatter(deg_v, [idx], ones)
            return carry

        kc = jnp.where(c == 0, k0, k1)
        lax.fori_loop(0, kc * groups, body, 0)
        pltpu.sync_copy(deg_v, out_hbm.at[wid])

    return deg_kernel


NBUF = 2
PAD_CHUNKS = NBUF


def _make_scatter_kernel(k0, k1, d):
    kt = max(k0, k1) + PAD_CHUNKS

    @functools.partial(
        pl.kernel,
        mesh=_sc_mesh(),
        out_type=jax.ShapeDtypeStruct((NC, N_PAD, d), jnp.float32),
        compiler_params=pltpu.CompilerParams(needs_layout_passes=False,
                                             use_tc_tiling_on_sc=False),
        scratch_types=[
            pltpu.VMEM((kt, CHUNK), jnp.int32),
            pltpu.VMEM((kt, CHUNK), jnp.int32),
            pltpu.VMEM((NBUF, CHUNK, d), jnp.float32),
            pltpu.VMEM_SHARED((N_PAD, d), jnp.float32),
        ] + [pltpu.SemaphoreType.DMA] * NBUF,
    )
    def sc_kernel(g_hbm, src_hbm, dst_hbm, zero_hbm, out_hbm,
                  src_v, dst_v, rows_v, acc_sh, *gsem):
        c = lax.axis_index("c")
        s = lax.axis_index("s")
        wid = c * NS + s
        row0 = s * RPT

        pltpu.sync_copy(zero_hbm.at[pl.ds(row0, RPT)],
                        acc_sh.at[pl.ds(row0, RPT)])
        pltpu.sync_copy(src_hbm.at[wid], src_v)
        pltpu.sync_copy(dst_hbm.at[wid], dst_v)
        plsc.subcore_barrier()

        for b in range(NBUF):
            pltpu.async_copy(g_hbm.at[src_v.at[b]], rows_v.at[b], gsem[b],
                             priority=1)

        def body(i, carry):
            j0 = NBUF * i
            for b in range(NBUF):
                jb = j0 + b
                pltpu.make_async_copy(g_hbm.at[src_v.at[jb]], rows_v.at[b],
                                      gsem[b]).wait()
                pltpu.sync_copy(rows_v.at[b], acc_sh.at[dst_v.at[jb]],
                                add=True)
                pltpu.async_copy(g_hbm.at[src_v.at[jb + NBUF]], rows_v.at[b],
                                 gsem[b], priority=1)
            return carry

        n_rounds = jnp.where(c == 0, k0 // NBUF, k1 // NBUF)
        lax.fori_loop(0, n_rounds, body, 0)
        for b in range(NBUF):
            pltpu.make_async_copy(g_hbm.at[src_v.at[b]], rows_v.at[b],
                                  gsem[b]).wait()

        plsc.subcore_barrier()
        pltpu.sync_copy(acc_sh.at[pl.ds(row0, RPT)],
                        out_hbm.at[c, pl.ds(row0, RPT)])

    return sc_kernel


RB = 1024
GRID = -(-N_NODES // RB)


def _tc_encode1(deg_parts, x, w1):
    def body(dp_ref, x_ref, w_ref, g_ref, dis_ref):
        deg = jnp.sum(dp_ref[...], axis=0) + 1.0
        dis = lax.rsqrt(deg)[:, None]
        h = jnp.dot(x_ref[...], w_ref[...],
                    preferred_element_type=jnp.float32)
        g_ref[...] = dis * h
        dis_ref[...] = dis

    return pl.pallas_call(
        body,
        grid=(GRID,),
        in_specs=[
            pl.BlockSpec((NW, RB), lambda i: (0, i)),
            pl.BlockSpec((RB, 128), lambda i: (i, 0)),
            pl.BlockSpec((128, 64), lambda i: (0, 0)),
        ],
        out_specs=[
            pl.BlockSpec((RB, 64), lambda i: (i, 0)),
            pl.BlockSpec((RB, 1), lambda i: (i, 0)),
        ],
        out_shape=[
            jax.ShapeDtypeStruct((N_NODES, 64), jnp.float32),
            jax.ShapeDtypeStruct((N_NODES, 1), jnp.float32),
        ],
    )(deg_parts, x, w1)


def _tc_encode2(disv, acc1, g1, b1, w2):
    def body(dis_ref, acc_ref, g1_ref, b_ref, w_ref, out_ref):
        dis = dis_ref[...]
        ssum = acc_ref[0] + acc_ref[1] + g1_ref[...]
        h1 = jnp.maximum(dis * ssum + b_ref[...], 0.0)
        h2 = jnp.dot(h1, w_ref[...], preferred_element_type=jnp.float32)
        out_ref[...] = dis * h2

    return pl.pallas_call(
        body,
        grid=(GRID,),
        in_specs=[
            pl.BlockSpec((RB, 1), lambda i: (i, 0)),
            pl.BlockSpec((NC, RB, 64), lambda i: (0, i, 0)),
            pl.BlockSpec((RB, 64), lambda i: (i, 0)),
            pl.BlockSpec((1, 64), lambda i: (0, 0)),
            pl.BlockSpec((64, 32), lambda i: (0, 0)),
        ],
        out_specs=pl.BlockSpec((RB, 32), lambda i: (i, 0)),
        out_shape=jax.ShapeDtypeStruct((N_NODES, 32), jnp.float32),
    )(disv, acc1, g1, b1, w2)


def _tc_decode(disv, acc2, g2, b2, wd1, bd1, wd2, bd2):
    def body(dis_ref, acc_ref, g2_ref, b2_ref, wd1_ref, bd1_ref, wd2_ref,
             bd2_ref, out_ref):
        dis = dis_ref[...]
        ssum = acc_ref[0] + acc_ref[1] + g2_ref[...]
        z = jnp.maximum(dis * ssum + b2_ref[...], 0.0)
        dlat = jnp.dot(z, wd1_ref[...], preferred_element_type=jnp.float32)
        dlat = jnp.maximum(dlat + bd1_ref[...], 0.0)
        logits = jnp.dot(dlat, wd2_ref[...],
                         preferred_element_type=jnp.float32)
        out_ref[...] = jax.nn.sigmoid(logits + bd2_ref[...])

    return pl.pallas_call(
        body,
        grid=(GRID,),
        in_specs=[
            pl.BlockSpec((RB, 1), lambda i: (i, 0)),
            pl.BlockSpec((NC, RB, 32), lambda i: (0, i, 0)),
            pl.BlockSpec((RB, 32), lambda i: (i, 0)),
            pl.BlockSpec((1, 32), lambda i: (0, 0)),
            pl.BlockSpec((32, 64), lambda i: (0, 0)),
            pl.BlockSpec((1, 64), lambda i: (0, 0)),
            pl.BlockSpec((64, 128), lambda i: (0, 0)),
            pl.BlockSpec((1, 128), lambda i: (0, 0)),
        ],
        out_specs=pl.BlockSpec((RB, 128), lambda i: (i, 0)),
        out_shape=jax.ShapeDtypeStruct((N_NODES, 128), jnp.float32),
    )(disv, acc2, g2, b2, wd1, bd1, wd2, bd2)


def kernel(x, edge_index, W1, b1, W2, b2, Wd1, bd1, Wd2, bd2):
    e = edge_index.shape[1]
    src = edge_index[0].astype(jnp.int32)
    dst = edge_index[1].astype(jnp.int32)

    chunks_tot = -(-e // CHUNK)
    k0 = int(round(chunks_tot * CORE0_FRAC / NS))
    k0 = max(NBUF, -(-k0 // NBUF) * NBUF)
    k1 = -(-(chunks_tot - NS * k0) // (NS * NBUF)) * NBUF
    k1 = max(NBUF, k1)
    kt = max(k0, k1) + PAD_CHUNKS

    cap0 = NS * k0 * CHUNK
    cap_tot = NS * (k0 + k1) * CHUNK
    src_f = jnp.concatenate([src, jnp.zeros((cap_tot - e,), jnp.int32)])
    dst_f = jnp.concatenate([dst, jnp.full((cap_tot - e,), N_NODES,
                                           jnp.int32)])

    def shard(flat, kc, fill):
        m = flat.reshape(NS, kc, CHUNK)
        padc = jnp.full((NS, kt - kc, CHUNK), fill, jnp.int32)
        return jnp.concatenate([m, padc], axis=1)

    src_all = jnp.concatenate([shard(src_f[:cap0], k0, 0),
                               shard(src_f[cap0:], k1, 0)], axis=0)
    dst_all = jnp.concatenate([shard(dst_f[:cap0], k0, N_NODES),
                               shard(dst_f[cap0:], k1, N_NODES)], axis=0)

    deg_parts = _make_deg_kernel(kt, k0, k1)(dst_all)

    g1, disv = _tc_encode1(deg_parts, x, W1)

    zeros64 = jnp.zeros((N_PAD, 64), jnp.float32)
    acc1 = _make_scatter_kernel(k0, k1, 64)(g1, src_all, dst_all, zeros64)

    g2 = _tc_encode2(disv, acc1[:, :N_NODES, :], g1, b1.reshape(1, 64), W2)

    zeros32 = jnp.zeros((N_PAD, 32), jnp.float32)
    acc2 = _make_scatter_kernel(k0, k1, 32)(g2, src_all, dst_all, zeros32)

    return _tc_decode(disv, acc2[:, :N_NODES, :], g2, b2.reshape(1, 32),
                      Wd1, bd1.reshape(1, 64), Wd2, bd2.reshape(1, 128))
